# baseline (speedup 1.0000x reference)
import functools

import jax
import jax.numpy as jnp
from jax import lax
from jax.experimental import pallas as pl
from jax.experimental.pallas import tpu as pltpu
from jax.experimental.pallas import tpu_sc as plsc

D = 64
SCALE = 8.0
HIST = 200
TAIL = HIST - 128


def _body(t0_hbm, t1_hbm, table_hbm, out_hbm, idx0_v, idx1_v, rows_v, sem,
          *, rows_per_w, nc):
    wid = lax.axis_index("s") * nc + lax.axis_index("c")
    b0 = wid * rows_per_w
    pltpu.sync_copy(t0_hbm.at[pl.ds(b0, rows_per_w)], idx0_v)
    pltpu.sync_copy(t1_hbm.at[pl.ds(b0, rows_per_w)], idx1_v)

    def row_body(j, carry):
        cp0 = pltpu.async_copy(
            table_hbm.at[idx0_v.at[j]], rows_v.at[pl.ds(0, 128)], sem)
        cp1 = pltpu.async_copy(
            table_hbm.at[idx1_v.at[j, pl.ds(0, TAIL)]],
            rows_v.at[pl.ds(128, TAIL)], sem)
        cp0.wait()
        cp1.wait()

        def scale_body(r, carry2):
            for k in range(D // 16):
                sl = (r, pl.ds(k * 16, 16))
                rows_v[sl] = rows_v[sl] * SCALE
            return carry2

        lax.fori_loop(0, HIST, scale_body, 0, unroll=4)

        pltpu.sync_copy(rows_v, out_hbm.at[b0 + j])
        return carry

    lax.fori_loop(0, rows_per_w, row_body, 0)


def kernel(tokens, table):
    batch, hist = tokens.shape
    assert hist == HIST
    info = plsc.get_sparse_core_info()
    nc, ns = info.num_cores, info.num_subcores
    nw = nc * ns
    rows_per_w = batch // nw

    tok = tokens.astype(jnp.int32)
    t0 = tok[:, :128]
    t1 = jnp.pad(tok[:, 128:], ((0, 0), (0, 128 - TAIL)))

    mesh = plsc.VectorSubcoreMesh(core_axis_name="c", subcore_axis_name="s")

    run = pl.kernel(
        functools.partial(_body, rows_per_w=rows_per_w, nc=nc),
        mesh=mesh,
        out_type=jax.ShapeDtypeStruct((batch, HIST, D), jnp.float32),
        scratch_types=[
            pltpu.VMEM((rows_per_w, 128), jnp.int32),
            pltpu.VMEM((rows_per_w, 128), jnp.int32),
            pltpu.VMEM((HIST, D), jnp.float32),
            pltpu.SemaphoreType.DMA,
        ],
        compiler_params=pltpu.CompilerParams(use_tc_tiling_on_sc=False),
    )
    return run(t0, t1, table)
